# baseline (speedup 1.0000x reference)
import functools

import jax
import jax.numpy as jnp
from jax import lax
from jax.experimental import pallas as pl
from jax.experimental.pallas import tpu as pltpu
from jax.experimental.pallas import tpu_sc as plsc

_BM = 256
_BN = 2048
_SC_CHUNK = 144


def _argmin_kernel(nj, bn, n_e, loss_scale, a_ref, z_ref, cb_ref, c_ref, idx_ref,
                   loss_ref):
    zb = z_ref[...]
    av = a_ref[...]
    bm = zb.shape[0]

    def step(jj, carry):
        mv, mi = carry
        cb = cb_ref[pl.ds(jj * bn, bn), :]
        cv = c_ref[pl.ds(jj * bn, bn)]
        mm = lax.dot_general(zb, cb, (((1,), (1,)), ((), ())),
                             preferred_element_type=jnp.float32)
        d = (av[:, None] - 2.0 * mm) + cv[None, :]
        tmin = jnp.min(d, axis=1)
        ids = lax.broadcasted_iota(jnp.int32, (bm, bn), 1)
        tidx = jnp.min(jnp.where(d == tmin[:, None], ids, n_e), axis=1) + jj * bn
        better = tmin < mv
        return jnp.where(better, tmin, mv), jnp.where(better, tidx, mi)

    mv0 = jnp.full((bm,), jnp.inf, dtype=jnp.float32)
    mi0 = jnp.zeros((bm,), dtype=jnp.int32)
    mv, mi = lax.fori_loop(0, nj, step, (mv0, mi0))
    idx_ref[...] = mi
    loss_ref[...] = mv * jnp.float32(loss_scale)


def _tc_argmin_call(zf, codebook, a, c):
    n, e = zf.shape
    n_e = codebook.shape[0]
    bm, bn = _BM, _BN
    ni, nj = n // bm, n_e // bn
    return pl.pallas_call(
        functools.partial(_argmin_kernel, nj, bn, n_e, 1.25 / e),
        grid=(ni,),
        in_specs=[
            pl.BlockSpec((bm,), lambda i: (i,)),
            pl.BlockSpec((bm, e), lambda i: (i, 0)),
            pl.BlockSpec((n_e, e), lambda i: (0, 0)),
            pl.BlockSpec((n_e,), lambda i: (0,)),
        ],
        out_specs=[
            pl.BlockSpec((bm,), lambda i: (i,)),
            pl.BlockSpec((bm,), lambda i: (i,)),
        ],
        out_shape=[
            jax.ShapeDtypeStruct((n,), jnp.int32),
            jax.ShapeDtypeStruct((n,), jnp.float32),
        ],
    )(a, zf, codebook, c)


def _sc_gather_call(codebook, idx):
    info = plsc.get_sparse_core_info()
    nc, ns = info.num_cores, info.num_subcores
    nw = nc * ns
    n = idx.shape[0]
    v, d = codebook.shape
    b_per_w = n // nw
    chunk = _SC_CHUNK
    nchunks = b_per_w // chunk
    mesh = plsc.VectorSubcoreMesh(core_axis_name="c", subcore_axis_name="s")

    @functools.partial(
        pl.kernel,
        mesh=mesh,
        out_type=jax.ShapeDtypeStruct((n, d), jnp.float32),
        scratch_types=[
            pltpu.VMEM((chunk,), jnp.int32),
            pltpu.VMEM((chunk, d), jnp.float32),
            pltpu.SemaphoreType.DMA,
        ],
    )
    def gather_k(table_hbm, idx_hbm, out_hbm, idx_v, rows_v, sem):
        wid = lax.axis_index("s") * nc + lax.axis_index("c")
        base = wid * b_per_w

        def body(k, carry):
            off = base + k * chunk
            pltpu.sync_copy(idx_hbm.at[pl.ds(off, chunk)], idx_v)
            pltpu.async_copy(table_hbm.at[idx_v], rows_v, sem).wait()
            pltpu.sync_copy(rows_v, out_hbm.at[pl.ds(off, chunk)])
            return carry

        lax.fori_loop(0, nchunks, body, 0)

    return gather_k(codebook, idx)


def kernel(z, codebook):
    b, t, e = z.shape
    zf = z.reshape(-1, e)
    a = jnp.sum(zf ** 2, axis=1)
    c = jnp.sum(codebook ** 2, axis=1)
    idx, loss = _tc_argmin_call(zf, codebook, a, c)
    zq = _sc_gather_call(codebook, idx)
    z_q_out = z + (zq.reshape(z.shape) - z)
    return z_q_out, loss.reshape(b, t), idx.reshape(b, t, 1)


# speedup vs baseline: 1.2422x; 1.2422x over previous
import functools

import jax
import jax.numpy as jnp
from jax import lax
from jax.experimental import pallas as pl
from jax.experimental.pallas import tpu as pltpu
from jax.experimental.pallas import tpu_sc as plsc

_BM = 256
_BN = 4096
_SC_CHUNK = 144


def _argmin_kernel(nj, bn, n_e, loss_scale, a_ref, z_ref, cb_ref, c_ref, idx_ref,
                   loss_ref):
    zb = z_ref[...] * jnp.float32(-2.0)
    av = a_ref[...]
    bm = zb.shape[0]
    ids = lax.broadcasted_iota(jnp.int32, (bm, bn), 1).astype(jnp.float32)

    def step(jj, carry):
        mv, mi = carry
        cb = cb_ref[pl.ds(jj * bn, bn), :]
        cv = c_ref[pl.ds(jj * bn, bn)]
        mm = lax.dot_general(zb, cb, (((1,), (1,)), ((), ())),
                             preferred_element_type=jnp.float32)
        d = (av[:, None] + mm) + cv[None, :]
        tmin = jnp.min(d, axis=1)
        tidx_f = jnp.min(jnp.where(d == tmin[:, None], ids, jnp.float32(n_e)),
                         axis=1)
        tidx = tidx_f.astype(jnp.int32) + jj * bn
        better = tmin < mv
        return jnp.where(better, tmin, mv), jnp.where(better, tidx, mi)

    mv0 = jnp.full((bm,), jnp.inf, dtype=jnp.float32)
    mi0 = jnp.zeros((bm,), dtype=jnp.int32)
    mv, mi = lax.fori_loop(0, nj, step, (mv0, mi0))
    idx_ref[...] = mi
    loss_ref[...] = mv * jnp.float32(loss_scale)


def _tc_argmin_call(zf, codebook, a, c):
    n, e = zf.shape
    n_e = codebook.shape[0]
    bm, bn = _BM, _BN
    ni, nj = n // bm, n_e // bn
    return pl.pallas_call(
        functools.partial(_argmin_kernel, nj, bn, n_e, 1.25 / e),
        grid=(ni,),
        in_specs=[
            pl.BlockSpec((bm,), lambda i: (i,)),
            pl.BlockSpec((bm, e), lambda i: (i, 0)),
            pl.BlockSpec((n_e, e), lambda i: (0, 0)),
            pl.BlockSpec((n_e,), lambda i: (0,)),
        ],
        out_specs=[
            pl.BlockSpec((bm,), lambda i: (i,)),
            pl.BlockSpec((bm,), lambda i: (i,)),
        ],
        out_shape=[
            jax.ShapeDtypeStruct((n,), jnp.int32),
            jax.ShapeDtypeStruct((n,), jnp.float32),
        ],
    )(a, zf, codebook, c)


def _sc_gather_call(codebook, idx):
    info = plsc.get_sparse_core_info()
    nc, ns = info.num_cores, info.num_subcores
    nw = nc * ns
    n = idx.shape[0]
    v, d = codebook.shape
    b_per_w = n // nw
    chunk = _SC_CHUNK
    nchunks = b_per_w // chunk
    mesh = plsc.VectorSubcoreMesh(core_axis_name="c", subcore_axis_name="s")

    @functools.partial(
        pl.kernel,
        mesh=mesh,
        out_type=jax.ShapeDtypeStruct((n, d), jnp.float32),
        scratch_types=[
            pltpu.VMEM((chunk,), jnp.int32),
            pltpu.VMEM((chunk, d), jnp.float32),
            pltpu.SemaphoreType.DMA,
        ],
    )
    def gather_k(table_hbm, idx_hbm, out_hbm, idx_v, rows_v, sem):
        wid = lax.axis_index("s") * nc + lax.axis_index("c")
        base = wid * b_per_w

        def body(k, carry):
            off = base + k * chunk
            pltpu.sync_copy(idx_hbm.at[pl.ds(off, chunk)], idx_v)
            pltpu.async_copy(table_hbm.at[idx_v], rows_v, sem).wait()
            pltpu.sync_copy(rows_v, out_hbm.at[pl.ds(off, chunk)])
            return carry

        lax.fori_loop(0, nchunks, body, 0)

    return gather_k(codebook, idx)


def kernel(z, codebook):
    b, t, e = z.shape
    zf = z.reshape(-1, e)
    a = jnp.sum(zf ** 2, axis=1)
    c = jnp.sum(codebook ** 2, axis=1)
    idx, loss = _tc_argmin_call(zf, codebook, a, c)
    zq = _sc_gather_call(codebook, idx)
    z_q_out = z + (zq.reshape(z.shape) - z)
    return z_q_out, loss.reshape(b, t), idx.reshape(b, t, 1)


# speedup vs baseline: 1.4379x; 1.1576x over previous
import functools

import jax
import jax.numpy as jnp
from jax import lax
from jax.experimental import pallas as pl
from jax.experimental.pallas import tpu as pltpu
from jax.experimental.pallas import tpu_sc as plsc

_BM = 256
_BN = 8192
_SC_CHUNK = 144


def _argmin_kernel(nj, bn, n_e, loss_scale, a_ref, z_ref, cb_ref, c_ref, idx_ref,
                   loss_ref):
    zb = z_ref[...] * jnp.float32(-2.0)
    av = a_ref[...]
    bm = zb.shape[0]
    ids = lax.broadcasted_iota(jnp.int32, (bm, bn), 1).astype(jnp.float32)

    def step(jj, carry):
        mv, mi = carry
        cb = cb_ref[pl.ds(jj * bn, bn), :]
        cv = c_ref[pl.ds(jj * bn, bn)]
        mm = lax.dot_general(zb, cb, (((1,), (1,)), ((), ())),
                             preferred_element_type=jnp.float32)
        d = (av[:, None] + mm) + cv[None, :]
        v, iv = d, ids
        h = bn // 2
        while h >= 128:
            lt = v[:, h:] < v[:, :h]
            v = jnp.where(lt, v[:, h:], v[:, :h])
            iv = jnp.where(lt, iv[:, h:], iv[:, :h])
            h //= 2
        tmin = jnp.min(v, axis=1)
        tidx_f = jnp.min(jnp.where(v == tmin[:, None], iv, jnp.float32(n_e)),
                         axis=1)
        tidx = tidx_f.astype(jnp.int32) + jj * bn
        better = tmin < mv
        return jnp.where(better, tmin, mv), jnp.where(better, tidx, mi)

    mv0 = jnp.full((bm,), jnp.inf, dtype=jnp.float32)
    mi0 = jnp.zeros((bm,), dtype=jnp.int32)
    mv, mi = lax.fori_loop(0, nj, step, (mv0, mi0))
    idx_ref[...] = mi
    loss_ref[...] = mv * jnp.float32(loss_scale)


def _tc_argmin_call(zf, codebook, a, c):
    n, e = zf.shape
    n_e = codebook.shape[0]
    bm, bn = _BM, _BN
    ni, nj = n // bm, n_e // bn
    return pl.pallas_call(
        functools.partial(_argmin_kernel, nj, bn, n_e, 1.25 / e),
        grid=(ni,),
        in_specs=[
            pl.BlockSpec((bm,), lambda i: (i,)),
            pl.BlockSpec((bm, e), lambda i: (i, 0)),
            pl.BlockSpec((n_e, e), lambda i: (0, 0)),
            pl.BlockSpec((n_e,), lambda i: (0,)),
        ],
        out_specs=[
            pl.BlockSpec((bm,), lambda i: (i,)),
            pl.BlockSpec((bm,), lambda i: (i,)),
        ],
        out_shape=[
            jax.ShapeDtypeStruct((n,), jnp.int32),
            jax.ShapeDtypeStruct((n,), jnp.float32),
        ],
    )(a, zf, codebook, c)


def _sc_gather_call(codebook, idx):
    info = plsc.get_sparse_core_info()
    nc, ns = info.num_cores, info.num_subcores
    nw = nc * ns
    n = idx.shape[0]
    v, d = codebook.shape
    b_per_w = n // nw
    chunk = _SC_CHUNK
    nchunks = b_per_w // chunk
    mesh = plsc.VectorSubcoreMesh(core_axis_name="c", subcore_axis_name="s")

    @functools.partial(
        pl.kernel,
        mesh=mesh,
        out_type=jax.ShapeDtypeStruct((n, d), jnp.float32),
        scratch_types=[
            pltpu.VMEM((chunk,), jnp.int32),
            pltpu.VMEM((chunk, d), jnp.float32),
            pltpu.SemaphoreType.DMA,
        ],
    )
    def gather_k(table_hbm, idx_hbm, out_hbm, idx_v, rows_v, sem):
        wid = lax.axis_index("s") * nc + lax.axis_index("c")
        base = wid * b_per_w

        def body(k, carry):
            off = base + k * chunk
            pltpu.sync_copy(idx_hbm.at[pl.ds(off, chunk)], idx_v)
            pltpu.async_copy(table_hbm.at[idx_v], rows_v, sem).wait()
            pltpu.sync_copy(rows_v, out_hbm.at[pl.ds(off, chunk)])
            return carry

        lax.fori_loop(0, nchunks, body, 0)

    return gather_k(codebook, idx)


def kernel(z, codebook):
    b, t, e = z.shape
    zf = z.reshape(-1, e)
    a = jnp.sum(zf ** 2, axis=1)
    c = jnp.sum(codebook ** 2, axis=1)
    idx, loss = _tc_argmin_call(zf, codebook, a, c)
    zq = _sc_gather_call(codebook, idx)
    return zq.reshape(z.shape), loss.reshape(b, t), idx.reshape(b, t, 1)


# speedup vs baseline: 1.5282x; 1.0628x over previous
import functools

import jax
import jax.numpy as jnp
from jax import lax
from jax.experimental import pallas as pl
from jax.experimental.pallas import tpu as pltpu
from jax.experimental.pallas import tpu_sc as plsc

_BM = 256
_BN = 8192
_SC_CHUNK = 144


def _argmin_kernel(nj, bn, n_e, loss_scale, a_ref, z_ref, cb_ref, c_ref,
                   idx_ref, loss_ref):
    zb = z_ref[...] * jnp.float32(-2.0)
    av = a_ref[...]
    bm = zb.shape[0]
    ids = lax.broadcasted_iota(jnp.int32, (bm, bn), 1).astype(jnp.float32)

    mv = mi = None
    for jj in range(nj):
        cb = cb_ref[pl.ds(jj * bn, bn), :]
        cv = c_ref[pl.ds(jj * bn, bn)]
        mm = lax.dot_general(zb, cb, (((1,), (1,)), ((), ())),
                             preferred_element_type=jnp.float32)
        d = (av[:, None] + mm) + cv[None, :]
        tmin = jnp.min(d, axis=1)
        tidx_f = jnp.min(jnp.where(d == tmin[:, None], ids, jnp.float32(n_e)),
                         axis=1)
        tidx = tidx_f.astype(jnp.int32) + jj * bn
        if jj == 0:
            mv, mi = tmin, tidx
        else:
            better = tmin < mv
            mv = jnp.where(better, tmin, mv)
            mi = jnp.where(better, tidx, mi)
    idx_ref[...] = mi
    loss_ref[...] = mv * jnp.float32(loss_scale)


def _tc_argmin_call(zf, codebook, a, c):
    n, e = zf.shape
    n_e = codebook.shape[0]
    bm, bn = _BM, _BN
    ni, nj = n // bm, n_e // bn
    return pl.pallas_call(
        functools.partial(_argmin_kernel, nj, bn, n_e, 1.25 / e),
        grid=(ni,),
        in_specs=[
            pl.BlockSpec((bm,), lambda i: (i,)),
            pl.BlockSpec((bm, e), lambda i: (i, 0)),
            pl.BlockSpec((n_e, e), lambda i: (0, 0)),
            pl.BlockSpec((n_e,), lambda i: (0,)),
        ],
        out_specs=[
            pl.BlockSpec((bm,), lambda i: (i,)),
            pl.BlockSpec((bm,), lambda i: (i,)),
        ],
        out_shape=[
            jax.ShapeDtypeStruct((n,), jnp.int32),
            jax.ShapeDtypeStruct((n,), jnp.float32),
        ],
    )(a, zf, codebook, c)


def _sc_gather_call(codebook, idx):
    info = plsc.get_sparse_core_info()
    nc, ns = info.num_cores, info.num_subcores
    nw = nc * ns
    n = idx.shape[0]
    v, d = codebook.shape
    b_per_w = n // nw
    chunk = _SC_CHUNK
    nchunks = b_per_w // chunk
    mesh = plsc.VectorSubcoreMesh(core_axis_name="c", subcore_axis_name="s")

    @functools.partial(
        pl.kernel,
        mesh=mesh,
        out_type=jax.ShapeDtypeStruct((n, d), jnp.float32),
        scratch_types=[
            pltpu.VMEM((chunk,), jnp.int32),
            pltpu.VMEM((chunk, d), jnp.float32),
            pltpu.SemaphoreType.DMA,
        ],
    )
    def gather_k(table_hbm, idx_hbm, out_hbm, idx_v, rows_v, sem):
        wid = lax.axis_index("s") * nc + lax.axis_index("c")
        base = wid * b_per_w

        def body(k, carry):
            off = base + k * chunk
            pltpu.sync_copy(idx_hbm.at[pl.ds(off, chunk)], idx_v)
            pltpu.async_copy(table_hbm.at[idx_v], rows_v, sem).wait()
            pltpu.sync_copy(rows_v, out_hbm.at[pl.ds(off, chunk)])
            return carry

        lax.fori_loop(0, nchunks, body, 0)

    return gather_k(codebook, idx)


def kernel(z, codebook):
    b, t, e = z.shape
    zf = z.reshape(-1, e)
    a = jnp.sum(zf ** 2, axis=1)
    c = jnp.sum(codebook ** 2, axis=1)
    idx, loss = _tc_argmin_call(zf, codebook, a, c)
    zq = _sc_gather_call(codebook, idx)
    return zq.reshape(z.shape), loss.reshape(b, t), idx.reshape(b, t, 1)
